# Initial kernel scaffold; baseline (speedup 1.0000x reference)
import jax
import jax.numpy as jnp
from jax.experimental import pallas as pl


def kernel(user_ids, item_ids, W, U, W1, b1, W2, b2):
    raise NotImplementedError("write your pallas kernel here")


# speedup vs baseline: 3.6493x; 3.6493x over previous
import functools

import jax
import jax.numpy as jnp
from jax import lax
from jax.experimental import pallas as pl
from jax.experimental.pallas import tpu as pltpu
from jax.experimental.pallas import tpu_sc as plsc

_B = 16384
_K = 128
_H = 256
_NC = 2
_NS = 16
_NW = _NC * _NS
_BPW = _B // _NW
_CHUNK = 128
_NCHUNK = _BPW // _CHUNK


def _vc_body(w1_ref, w2_ref, b1_ref, b2_ref, out_ref):
    v = jnp.dot(w2_ref[...], w1_ref[...], preferred_element_type=jnp.float32)
    c = jnp.sum(w2_ref[...] * b1_ref[...]) + b2_ref[0, 0]
    out_ref[:, :256] = v
    out_ref[:, 256:] = jnp.full((1, 128), c, jnp.float32)


def _sc_body(uid_hbm, iid_hbm, w_hbm, u_hbm, vc_hbm, out_hbm,
             uid_v, iid_v, wbuf, ubuf, obuf, vbuf, sem_w, sem_u):
    wid = lax.axis_index("s") * _NC + lax.axis_index("c")
    base = wid * _BPW
    pltpu.sync_copy(uid_hbm.at[pl.ds(base, _BPW)], uid_v)
    pltpu.sync_copy(iid_hbm.at[pl.ds(base, _BPW)], iid_v)
    pltpu.sync_copy(vc_hbm, vbuf)
    vw = [vbuf[pl.ds(16 * j, 16)] for j in range(8)]
    vu = [vbuf[pl.ds(128 + 16 * j, 16)] for j in range(8)]
    cval = vbuf[pl.ds(256, 16)][0]
    lanes = lax.iota(jnp.int32, 16)
    bfly = [lanes ^ 8, lanes ^ 4, lanes ^ 2, lanes ^ 1]
    dnums = lax.GatherDimensionNumbers(
        offset_dims=(), collapsed_slice_dims=(0,), start_index_map=(0,))

    def shuf(x, idx):
        return lax.gather(x, idx[:, None], dnums, (1,),
                          mode=lax.GatherScatterMode.PROMISE_IN_BOUNDS)

    def chunk(g, carry):
        cw = pltpu.async_copy(
            w_hbm.at[uid_v.at[pl.ds(g * _CHUNK, _CHUNK)]], wbuf, sem_w)
        cu = pltpu.async_copy(
            u_hbm.at[iid_v.at[pl.ds(g * _CHUNK, _CHUNK)]], ubuf, sem_u)
        cw.wait()
        cu.wait()

        def group(t, inner):
            outv = jnp.zeros((16,), jnp.float32)
            for i in range(16):
                r = t * 16 + i
                acc = wbuf[r, pl.ds(0, 16)] * vw[0]
                for j in range(1, 8):
                    acc = acc + wbuf[r, pl.ds(16 * j, 16)] * vw[j]
                for j in range(8):
                    acc = acc + ubuf[r, pl.ds(16 * j, 16)] * vu[j]
                for p in bfly:
                    acc = acc + shuf(acc, p)
                outv = jnp.where(lanes == i, acc + cval, outv)
            outv = jnp.clip(outv, 0.5, 5.0)
            obuf[pl.ds(g * _CHUNK + t * 16, 16)] = outv
            return inner

        lax.fori_loop(0, _CHUNK // 16, group, 0)
        return carry

    lax.fori_loop(0, _NCHUNK, chunk, 0)
    pltpu.sync_copy(obuf, out_hbm.at[pl.ds(base, _BPW)])


def kernel(user_ids, item_ids, W, U, W1, b1, W2, b2):
    uid = user_ids.astype(jnp.int32)
    iid = item_ids.astype(jnp.int32)

    vc = pl.pallas_call(
        _vc_body,
        out_shape=jax.ShapeDtypeStruct((1, 384), jnp.float32),
    )(W1, W2, b1.reshape(1, _H), b2.reshape(1, 1))
    vc_flat = vc.reshape(384)

    sc = functools.partial(
        pl.kernel,
        mesh=plsc.VectorSubcoreMesh(core_axis_name="c", subcore_axis_name="s"),
        out_type=jax.ShapeDtypeStruct((_B,), jnp.float32),
        scratch_types=[
            pltpu.VMEM((_BPW,), jnp.int32),
            pltpu.VMEM((_BPW,), jnp.int32),
            pltpu.VMEM((_CHUNK, _K), jnp.float32),
            pltpu.VMEM((_CHUNK, _K), jnp.float32),
            pltpu.VMEM((_BPW,), jnp.float32),
            pltpu.VMEM((384,), jnp.float32),
            pltpu.SemaphoreType.DMA,
            pltpu.SemaphoreType.DMA,
        ],
    )(_sc_body)
    return sc(uid, iid, W, U, vc_flat)
